# baseline (speedup 1.0000x reference)
import functools

import jax
import jax.numpy as jnp
from jax import lax
from jax.experimental import pallas as pl
from jax.experimental.pallas import tpu as pltpu
from jax.experimental.pallas import tpu_sc as plsc

_N = 5000
_D = _H = _W = 192
_NROW = 3 * _D * _H
_WHI = _W - 128
_NHI = 3 * _D * _H * _WHI

_NC = 2
_NS = 16
_NW = _NC * _NS
_CHUNK = 256
_NPAD = _NW * _CHUNK
_G = _CHUNK // 16
_NIDX = 6 * _CHUNK
_B = 128
_NB = _NIDX // _B

_mesh = plsc.VectorSubcoreMesh(core_axis_name="c", subcore_axis_name="s")


@functools.partial(
    pl.kernel,
    mesh=_mesh,
    compiler_params=pltpu.CompilerParams(needs_layout_passes=False),
    out_type=jax.ShapeDtypeStruct((3 * _NPAD,), jnp.float32),
    scratch_types=[
        pltpu.VMEM((3 * _CHUNK,), jnp.float32),
        pltpu.VMEM((3 * _CHUNK,), jnp.float32),
        pltpu.VMEM((3 * _CHUNK,), jnp.float32),
        pltpu.VMEM((128,), jnp.float32),
        pltpu.VMEM((_NIDX,), jnp.int32),
        pltpu.VMEM((_NIDX,), jnp.int32),
        pltpu.VMEM((_NIDX,), jnp.int32),
        pltpu.VMEM((_NIDX,), jnp.float32),
        pltpu.VMEM((_B, 128), jnp.float32),
        pltpu.VMEM((_B, 128), jnp.float32),
        pltpu.VMEM((_NIDX,), jnp.float32),
        pltpu.VMEM((3 * _CHUNK,), jnp.float32),
        pltpu.SemaphoreType.DMA,
        pltpu.SemaphoreType.DMA,
        pltpu.SemaphoreType.DMA,
    ],
)
def _trl_sc(fix_hbm, mov_hbm, field_hbm, hi_hbm, sp_hbm, out_hbm,
            mvr, fvr, mv, spv, rowv, colv, hidxv, hivalsv, buf0, buf1,
            valsv, ov, sem0, sem1, semi):
    wid = lax.axis_index("s") * _NC + lax.axis_index("c")
    base = wid * 3 * _CHUNK
    field2 = field_hbm.reshape(_NROW, _W)
    bufs = (buf0, buf1)
    sems = (sem0, sem1)

    in_copies = [
        pltpu.async_copy(mov_hbm.at[pl.ds(base, 3 * _CHUNK)], mvr, semi),
        pltpu.async_copy(fix_hbm.at[pl.ds(base, 3 * _CHUNK)], fvr, semi),
        pltpu.async_copy(sp_hbm, spv, semi),
    ]
    for cp in in_copies:
        cp.wait()

    lane3 = lax.iota(jnp.int32, 16) * 3
    lane = lax.iota(jnp.int32, 16)

    for i in range(_G):
        ix = lane3 + i * 48
        x = plsc.load_gather(mvr, [ix])
        y = plsc.load_gather(mvr, [ix + 1])
        z = plsc.load_gather(mvr, [ix + 2])
        mv[pl.ds(i * 16, 16)] = x
        mv[pl.ds(_CHUNK + i * 16, 16)] = y
        mv[pl.ds(2 * _CHUNK + i * 16, 16)] = z
        xf = x.astype(jnp.int32)
        yf = y.astype(jnp.int32)
        zf = z.astype(jnp.int32)
        xc = jnp.where(x > xf.astype(jnp.float32), xf + 1, xf)
        yc = jnp.where(y > yf.astype(jnp.float32), yf + 1, yf)
        zc = jnp.where(z > zf.astype(jnp.float32), zf + 1, zf)
        row_f = xf * _H + yf
        row_c = xc * _H + yc
        zero = jnp.zeros((16,), jnp.int32)
        for ch in range(3):
            sf = ch * _CHUNK + i * 16
            sc_ = (3 + ch) * _CHUNK + i * 16
            rowv[pl.ds(sf, 16)] = row_f + ch * (_D * _H)
            rowv[pl.ds(sc_, 16)] = row_c + ch * (_D * _H)
            colv[pl.ds(sf, 16)] = jnp.where(zf < 128, zf, zero)
            colv[pl.ds(sc_, 16)] = jnp.where(zc < 128, zc, zero)
            hidxv[pl.ds(sf, 16)] = jnp.where(
                zf >= 128, (row_f + ch * (_D * _H)) * _WHI + zf - 128, zero)
            hidxv[pl.ds(sc_, 16)] = jnp.where(
                zc >= 128, (row_c + ch * (_D * _H)) * _WHI + zc - 128, zero)

    hi_copies = [
        pltpu.async_copy(hi_hbm.at[hidxv.at[pl.ds(g * _B, _B)]],
                         hivalsv.at[pl.ds(g * _B, _B)], semi)
        for g in range(_NB)
    ]

    def fire(b):
        return pltpu.async_copy(
            field2.at[rowv.at[pl.ds(b * _B, _B)], pl.ds(0, 128)],
            bufs[b % 2], sems[b % 2])

    handles = [fire(0), fire(1)]
    for b in range(_NB):
        handles[b % 2].wait()
        for g in range(_B // 16):
            s = b * _B + g * 16
            vals = plsc.load_gather(
                bufs[b % 2], [lane + g * 16, colv[pl.ds(s, 16)]])
            valsv[pl.ds(s, 16)] = vals
        if b + 2 < _NB:
            handles[b % 2] = fire(b + 2)
    for cp in hi_copies:
        cp.wait()

    for ch in range(3):
        sp = spv[pl.ds(ch * 16, 16)]
        for i in range(_G):
            o = ch * _CHUNK + i * 16
            oc = 3 * _CHUNK + o
            z = mv[pl.ds(2 * _CHUNK + i * 16, 16)]
            zf = z.astype(jnp.int32)
            zc = jnp.where(z > zf.astype(jnp.float32), zf + 1, zf)
            f = jnp.where(zf < 128, valsv[pl.ds(o, 16)],
                          hivalsv[pl.ds(o, 16)])
            c = jnp.where(zc < 128, valsv[pl.ds(oc, 16)],
                          hivalsv[pl.ds(oc, 16)])
            fx = plsc.load_gather(fvr, [lane3 + i * 48 + ch])
            disp = (f + c) * 0.5
            res = (mv[pl.ds(o, 16)] + disp - fx) * sp
            plsc.store_scatter(ov, [lane3 + i * 48 + ch], res)
    pltpu.sync_copy(ov, out_hbm.at[pl.ds(base, 3 * _CHUNK)])


def kernel(fixed_landmarks, moving_landmarks, displacement_field,
           fixed_spacing, moving_spacing):
    pad = jnp.zeros((3 * _NPAD - 3 * _N,), jnp.float32)
    mov_flat = jnp.concatenate([moving_landmarks.reshape(3 * _N), pad])
    fix_flat = jnp.concatenate([fixed_landmarks.reshape(3 * _N), pad])
    hi_flat = displacement_field[:, :, :, :, 128:].reshape(_NHI)
    sp_b = jnp.concatenate([
        jnp.broadcast_to(moving_spacing.reshape(3, 1), (3, 16)).reshape(48),
        jnp.zeros((80,), jnp.float32),
    ])
    out_flat = _trl_sc(fix_flat, mov_flat, displacement_field, hi_flat, sp_b)
    return out_flat[:3 * _N].reshape(_N, 3)


# speedup vs baseline: 3.5867x; 3.5867x over previous
import functools

import jax
import jax.numpy as jnp
from jax import lax
from jax.experimental import pallas as pl
from jax.experimental.pallas import tpu as pltpu
from jax.experimental.pallas import tpu_sc as plsc

_N = 5000
_D = _H = _W = 192
_HW = _H * _W
_CHS = _D * _H * _W

_NC = 2
_NS = 16
_NW = _NC * _NS
_CHUNK = 256
_NPAD = _NW * _CHUNK
_G = _CHUNK // 16
_NIDX = 6 * _CHUNK

_mesh = plsc.VectorSubcoreMesh(core_axis_name="c", subcore_axis_name="s")


@functools.partial(
    pl.kernel,
    mesh=_mesh,
    compiler_params=pltpu.CompilerParams(needs_layout_passes=False),
    out_type=jax.ShapeDtypeStruct((3 * _NPAD,), jnp.float32),
    scratch_types=[
        pltpu.VMEM((3 * _CHUNK,), jnp.float32),
        pltpu.VMEM((3 * _CHUNK,), jnp.float32),
        pltpu.VMEM((3 * _CHUNK,), jnp.float32),
        pltpu.VMEM((128,), jnp.float32),
        pltpu.VMEM((_NIDX,), jnp.int32),
        pltpu.VMEM((_NIDX,), jnp.float32),
        pltpu.VMEM((3 * _CHUNK,), jnp.float32),
        pltpu.SemaphoreType.DMA,
    ],
)
def _trl_sc(fix_hbm, mov_hbm, field_hbm, sp_hbm, out_hbm,
            mvr, fvr, mv, spv, idxv, valsv, ov, sem):
    wid = lax.axis_index("s") * _NC + lax.axis_index("c")
    base = wid * 3 * _CHUNK

    in_copies = [
        pltpu.async_copy(mov_hbm.at[pl.ds(base, 3 * _CHUNK)], mvr, sem),
        pltpu.async_copy(fix_hbm.at[pl.ds(base, 3 * _CHUNK)], fvr, sem),
        pltpu.async_copy(sp_hbm, spv, sem),
    ]
    for cp in in_copies:
        cp.wait()

    lane3 = lax.iota(jnp.int32, 16) * 3

    copies = []
    for h in range(2):
        for j in range(8):
            i = h * 8 + j
            ix = lane3 + i * 48
            x = plsc.load_gather(mvr, [ix])
            y = plsc.load_gather(mvr, [ix + 1])
            z = plsc.load_gather(mvr, [ix + 2])
            mv[pl.ds(i * 16, 16)] = x
            mv[pl.ds(_CHUNK + i * 16, 16)] = y
            mv[pl.ds(2 * _CHUNK + i * 16, 16)] = z
            xf = x.astype(jnp.int32)
            yf = y.astype(jnp.int32)
            zf = z.astype(jnp.int32)
            xc = jnp.where(x > xf.astype(jnp.float32), xf + 1, xf)
            yc = jnp.where(y > yf.astype(jnp.float32), yf + 1, yf)
            zc = jnp.where(z > zf.astype(jnp.float32), zf + 1, zf)
            flat_f = xf * _HW + yf * _W + zf
            flat_c = xc * _HW + yc * _W + zc
            o = h * 768 + j * 16
            for ch in range(3):
                idxv[pl.ds(o + ch * 128, 16)] = flat_f + ch * _CHS
                idxv[pl.ds(o + (3 + ch) * 128, 16)] = flat_c + ch * _CHS
        copies.append(
            pltpu.async_copy(field_hbm.at[idxv.at[pl.ds(h * 768, 768)]],
                             valsv.at[pl.ds(h * 768, 768)], sem))
    for cp in copies:
        cp.wait()

    for ch in range(3):
        sp = spv[pl.ds(ch * 16, 16)]
        for i in range(_G):
            h, j = divmod(i, 8)
            o = h * 768 + j * 16
            f = valsv[pl.ds(o + ch * 128, 16)]
            c = valsv[pl.ds(o + (3 + ch) * 128, 16)]
            fx = plsc.load_gather(fvr, [lane3 + i * 48 + ch])
            disp = (f + c) * 0.5
            res = (mv[pl.ds(ch * _CHUNK + i * 16, 16)] + disp - fx) * sp
            plsc.store_scatter(ov, [lane3 + i * 48 + ch], res)
    pltpu.sync_copy(ov, out_hbm.at[pl.ds(base, 3 * _CHUNK)])


def kernel(fixed_landmarks, moving_landmarks, displacement_field,
           fixed_spacing, moving_spacing):
    pad = jnp.zeros((3 * _NPAD - 3 * _N,), jnp.float32)
    mov_flat = jnp.concatenate([moving_landmarks.reshape(3 * _N), pad])
    fix_flat = jnp.concatenate([fixed_landmarks.reshape(3 * _N), pad])
    field_flat = displacement_field.reshape(3 * _CHS)
    sp_b = jnp.concatenate([
        jnp.broadcast_to(moving_spacing.reshape(3, 1), (3, 16)).reshape(48),
        jnp.zeros((80,), jnp.float32),
    ])
    out_flat = _trl_sc(fix_flat, mov_flat, field_flat, sp_b)
    return out_flat[:3 * _N].reshape(_N, 3)
